# Initial kernel scaffold; baseline (speedup 1.0000x reference)
import jax
import jax.numpy as jnp
from jax.experimental import pallas as pl


def kernel(x, edge_index, W, b):
    raise NotImplementedError("write your pallas kernel here")


# speedup vs baseline: 15.1929x; 15.1929x over previous
import functools

import jax
import jax.numpy as jnp
from jax import lax
from jax.experimental import pallas as pl
from jax.experimental.pallas import tpu as pltpu
from jax.experimental.pallas import tpu_sc as plsc

N = 10000
D = 256
H = 128
E = 160000
NC, NS = 2, 16

_EC = E // (NC * NS)
_ROWS_A = 640


_NPAD = _ROWS_A * NS


def _deg_body(dst_hbm, degs0_hbm, degs1_hbm, dstv, ones_v, zb, deg_sh):
    c = lax.axis_index("c")
    s = lax.axis_index("s")

    def fill_ones(i, _):
        ones_v[pl.ds(i * 16, 16)] = jnp.ones((16,), jnp.float32)
        return 0

    lax.fori_loop(0, _EC // 16, fill_ones, 0)
    ones_v[pl.ds(_EC - 16, 16)] = jnp.ones((16,), jnp.float32)

    def fill_zero(i, _):
        zb[pl.ds(i * 16, 16)] = jnp.zeros((16,), jnp.float32)
        return 0

    lax.fori_loop(0, _ROWS_A // 16, fill_zero, 0)
    pltpu.sync_copy(zb, deg_sh.at[pl.ds(s * _ROWS_A, _ROWS_A)])
    plsc.subcore_barrier()
    off = c * (E // 2) + s * _EC
    pltpu.sync_copy(dst_hbm.at[pl.ds(off, _EC)], dstv)
    pltpu.sync_copy(ones_v, deg_sh.at[dstv], add=True)
    plsc.subcore_barrier()
    pltpu.sync_copy(deg_sh.at[pl.ds(s * _ROWS_A, _ROWS_A)], zb)

    @pl.when(c == 0)
    def _():
        pltpu.sync_copy(zb, degs0_hbm.at[pl.ds(s * _ROWS_A, _ROWS_A)])

    @pl.when(c == 1)
    def _():
        pltpu.sync_copy(zb, degs1_hbm.at[pl.ds(s * _ROWS_A, _ROWS_A)])


_deg_call = pl.kernel(
    _deg_body,
    out_type=[
        jax.ShapeDtypeStruct((_NPAD,), jnp.float32),
        jax.ShapeDtypeStruct((_NPAD,), jnp.float32),
    ],
    mesh=plsc.VectorSubcoreMesh(core_axis_name="c", subcore_axis_name="s"),
    scratch_types=[
        pltpu.VMEM((_EC,), jnp.int32),
        pltpu.VMEM((_EC,), jnp.float32),
        pltpu.VMEM((_ROWS_A,), jnp.float32),
        pltpu.VMEM_SHARED((_NPAD,), jnp.float32),
    ],
)

_RB = 1000


def _mm_body(x_ref, w_ref, d0_ref, d1_ref, y_ref, dis_ref):
    deg = d0_ref[...] + d1_ref[...] + 1.0
    dis = lax.rsqrt(deg)
    xw = jnp.dot(x_ref[...], w_ref[...], preferred_element_type=jnp.float32)
    y_ref[...] = xw * dis
    dis_ref[...] = dis


_mm_call = pl.pallas_call(
    _mm_body,
    grid=(2 * N // _RB,),
    in_specs=[
        pl.BlockSpec((_RB, D), lambda i: (i % 10, 0)),
        pl.BlockSpec((D, H), lambda i: (0, i // 10)),
        pl.BlockSpec((_RB, 1), lambda i: (i % 10, 0)),
        pl.BlockSpec((_RB, 1), lambda i: (i % 10, 0)),
    ],
    out_specs=[
        pl.BlockSpec((_RB, H), lambda i: (i, 0)),
        pl.BlockSpec((_RB, 1), lambda i: (i % 10, 0)),
    ],
    out_shape=[
        jax.ShapeDtypeStruct((2 * N, H), jnp.float32),
        jax.ShapeDtypeStruct((N, 1), jnp.float32),
    ],
)

_BE = 200
_ET = E // NS
_NB = _ET // _BE
_RT = 624


def _edge_body(y_hbm, src_hbm, dst_hbm, acc_hbm, idx_s, idx_d, rows_v, zbuf,
               acc_sh, sem):
    c = lax.axis_index("c")
    s = lax.axis_index("s")
    cn = c * N

    def fill_zero(i, _):
        zbuf[i // 8, pl.ds((i % 8) * 16, 16)] = jnp.zeros((16,), jnp.float32)
        return 0

    lax.fori_loop(0, 16 * (H // 16), fill_zero, 0)
    r0 = s * _RT
    nzb = jnp.where(s < 15, _RT // 16, 640 // 16)

    def zero_dma(j, _):
        pltpu.sync_copy(zbuf, acc_sh.at[pl.ds(r0 + 16 * j, 16)])
        return 0

    lax.fori_loop(0, nzb, zero_dma, 0)
    plsc.subcore_barrier()

    base_e = s * _ET

    def blk(bi, _):
        off = base_e + bi * _BE
        pltpu.sync_copy(src_hbm.at[pl.ds(c * E + off, _BE)], idx_s)
        pltpu.sync_copy(dst_hbm.at[pl.ds(off, _BE)], idx_d)
        pltpu.async_copy(y_hbm.at[idx_s], rows_v, sem).wait()
        pltpu.sync_copy(rows_v, acc_sh.at[idx_d], add=True)
        return 0

    lax.fori_loop(0, _NB, blk, 0)
    plsc.subcore_barrier()

    @pl.when(s < 15)
    def _():
        pltpu.sync_copy(acc_sh.at[pl.ds(r0, _RT)],
                        acc_hbm.at[pl.ds(cn + r0, _RT)])

    @pl.when(s == 15)
    def _():
        pltpu.sync_copy(acc_sh.at[pl.ds(15 * _RT, 640)],
                        acc_hbm.at[pl.ds(cn + 15 * _RT, 640)])


_edge_call = pl.kernel(
    _edge_body,
    out_type=jax.ShapeDtypeStruct((2 * N, H), jnp.float32),
    mesh=plsc.VectorSubcoreMesh(core_axis_name="c", subcore_axis_name="s"),
    scratch_types=[
        pltpu.VMEM((_BE,), jnp.int32),
        pltpu.VMEM((_BE,), jnp.int32),
        pltpu.VMEM((_BE, H), jnp.float32),
        pltpu.VMEM((16, H), jnp.float32),
        pltpu.VMEM_SHARED((N, H), jnp.float32),
        pltpu.SemaphoreType.DMA,
    ],
)


def _fin_body(al_ref, ar_ref, yl_ref, yr_ref, dis_ref, b_ref, out_ref):
    d = dis_ref[...]
    left = (al_ref[...] + yl_ref[...]) * d
    right = (ar_ref[...] + yr_ref[...]) * d
    out_ref[...] = jnp.concatenate([left, right], axis=1) + b_ref[...]


_fin_call = pl.pallas_call(
    _fin_body,
    grid=(N // _RB,),
    in_specs=[
        pl.BlockSpec((_RB, H), lambda i: (i, 0)),
        pl.BlockSpec((_RB, H), lambda i: (10 + i, 0)),
        pl.BlockSpec((_RB, H), lambda i: (i, 0)),
        pl.BlockSpec((_RB, H), lambda i: (10 + i, 0)),
        pl.BlockSpec((_RB, 1), lambda i: (i, 0)),
        pl.BlockSpec((1, D), lambda i: (0, 0)),
    ],
    out_specs=pl.BlockSpec((_RB, D), lambda i: (i, 0)),
    out_shape=jax.ShapeDtypeStruct((N, D), jnp.float32),
)


def kernel(x, edge_index, W, b):
    assert x.shape == (N, D) and W.shape == (D, D) and edge_index.shape == (2, E)
    src = edge_index[0].astype(jnp.int32)
    dst = edge_index[1].astype(jnp.int32)
    degs0, degs1 = _deg_call(dst)
    d0 = degs0[:N].reshape(N, 1)
    d1 = degs1[:N].reshape(N, 1)
    y, dis = _mm_call(x, W, d0, d1)
    src2 = jnp.concatenate([src, src + N])
    acc = _edge_call(y, src2, dst)
    out = _fin_call(acc, acc, y, y, dis, b.reshape(1, D))
    return out
